# baseline (speedup 1.0000x reference)
import jax
import jax.numpy as jnp
from jax.experimental import pallas as pl
from jax.experimental.pallas import tpu as pltpu

_C = 384
_H = 192
_M = _C - _H
_HW = 3136
_NB = 3


def _body(x_auto, x_hbm, w_ref, out_ref, buf, sem):
    b = pl.program_id(0)
    nb = pl.num_programs(0)

    @pl.when(b == 0)
    def _():
        for j in range(_NB - 1):
            pltpu.make_async_copy(
                x_hbm.at[j, pl.ds(_H, _M)], buf.at[j], sem.at[j]
            ).start()

    pre = b + _NB - 1
    slot_pre = jax.lax.rem(pre, _NB)

    @pl.when(pre < nb)
    def _():
        for j in range(_NB):

            @pl.when(slot_pre == j)
            def _():
                pltpu.make_async_copy(
                    x_hbm.at[pre, pl.ds(_H, _M)], buf.at[j], sem.at[j]
                ).start()

    slot = jax.lax.rem(b, _NB)
    for j in range(_NB):

        @pl.when(slot == j)
        def _():
            pltpu.make_async_copy(
                x_hbm.at[b, pl.ds(_H, _M)], buf.at[j], sem.at[j]
            ).wait()

    y1 = jnp.sum(x_auto[0], axis=1)
    y2 = jnp.sum(buf[slot], axis=1)
    yr = jnp.concatenate([y1.reshape(1, _H), y2.reshape(1, _M)], axis=1)
    iota = jax.lax.broadcasted_iota(jnp.int32, (1, _C), 1)
    scale = 1.0 / _HW
    w0 = w_ref[0] * scale
    w1 = w_ref[1] * scale
    w2 = w_ref[2] * scale
    yprev = jnp.where(iota == 0, 0.0, pltpu.roll(yr, 1, axis=1))
    ynext = jnp.where(iota == _C - 1, 0.0, pltpu.roll(yr, _C - 1, axis=1))
    s = w0 * yprev + w1 * yr + w2 * ynext
    cur = s
    for k in range(3):
        m = jnp.max(cur)
        idx_k = jnp.min(jnp.where(cur == m, iota, _C))
        row_a = x_auto[0, pl.ds(jnp.minimum(idx_k, _H - 1), 1)]
        row_m = buf[slot, pl.ds(jnp.maximum(idx_k - _H, 0), 1)]
        out_ref[0, pl.ds(k, 1)] = jnp.where(idx_k < _H, row_a, row_m)
        cur = jnp.where(iota == idx_k, -jnp.inf, cur)


@jax.jit
def kernel(x, w):
    b, c, h, wd = x.shape
    x3 = x.reshape(b, c, h * wd)
    out = pl.pallas_call(
        _body,
        grid=(b,),
        in_specs=[
            pl.BlockSpec((1, _H, h * wd), lambda i: (i, 0, 0)),
            pl.BlockSpec(memory_space=pl.ANY),
            pl.BlockSpec(memory_space=pltpu.SMEM),
        ],
        out_specs=pl.BlockSpec((1, 3, h * wd), lambda i: (i, 0, 0)),
        out_shape=jax.ShapeDtypeStruct((b, 3, h * wd), x.dtype),
        scratch_shapes=[
            pltpu.VMEM((_NB, _M, h * wd), jnp.float32),
            pltpu.SemaphoreType.DMA((_NB,)),
        ],
    )(x3, x3, w)
    return out.reshape(b, 3, h, wd)
